# baseline (speedup 1.0000x reference)
import functools

import jax
import jax.numpy as jnp
from jax import lax
from jax.experimental import pallas as pl
from jax.experimental.pallas import tpu as pltpu
from jax.experimental.pallas import tpu_sc as plsc

B = 16384
D = 64
HID = 16

_info = plsc.get_sparse_core_info()
NC, NS = _info.num_cores, _info.num_subcores
NW = NC * NS
B_PER_W = B // NW
CHUNK = 128
NCH = B_PER_W // CHUNK
L = 16

_mesh = plsc.VectorSubcoreMesh(core_axis_name="c", subcore_axis_name="s")


@functools.partial(
    pl.kernel,
    mesh=_mesh,
    out_type=jax.ShapeDtypeStruct((D, B), jnp.float32),
    scratch_types=[
        pltpu.VMEM((NCH, CHUNK), jnp.int32),
        pltpu.VMEM((B_PER_W, D), jnp.float32),
        pltpu.VMEM((D, B_PER_W), jnp.float32),
        pltpu.SemaphoreType.DMA,
    ],
    compiler_params=pltpu.CompilerParams(
        use_tc_tiling_on_sc=False, needs_layout_passes=False
    ),
)
def _sc_gather_t(idx_hbm, table_hbm, emb_t_hbm, idx_v, rows_v, out_v, sem):
    wid = lax.axis_index("s") * NC + lax.axis_index("c")
    jbase = wid * B_PER_W
    pltpu.sync_copy(idx_hbm.at[wid], idx_v)
    handles = []
    for k in range(NCH):
        handles.append(
            pltpu.async_copy(
                table_hbm.at[idx_v.at[k]],
                rows_v.at[pl.ds(k * CHUNK, CHUNK)],
                sem,
            )
        )
    for h in handles:
        h.wait()

    def body(jg, _):
        jv = jg * L + lax.iota(jnp.int32, L)
        for c in range(D):
            cv = jnp.full((L,), c, dtype=jnp.int32)
            val = plsc.load_gather(rows_v, [jv, cv])
            out_v[c, pl.ds(jg * L, L)] = val
        return 0

    lax.fori_loop(0, B_PER_W // L, body, 0)
    pltpu.sync_copy(out_v, emb_t_hbm.at[:, pl.ds(jbase, B_PER_W)])


def _mlp_body(w1t_ref, embt_ref, b1_ref, w2t_ref, b2_ref, out_ref):
    h = jnp.dot(w1t_ref[...], embt_ref[...], preferred_element_type=jnp.float32)
    h = jnp.maximum(h + b1_ref[...], 0.0)
    out_ref[...] = (
        jnp.dot(w2t_ref[...], h, preferred_element_type=jnp.float32) + b2_ref[...]
    )


_BJ = 2048


def _tc_mlp_t(w1t, embt, b1, w2t, b2):
    grid = (B // _BJ,)
    return pl.pallas_call(
        _mlp_body,
        grid=grid,
        in_specs=[
            pl.BlockSpec((HID, D), lambda j: (0, 0)),
            pl.BlockSpec((D, _BJ), lambda j: (0, j)),
            pl.BlockSpec((HID, 1), lambda j: (0, 0)),
            pl.BlockSpec((1, HID), lambda j: (0, 0)),
            pl.BlockSpec((1, 1), lambda j: (0, 0)),
        ],
        out_specs=pl.BlockSpec((1, _BJ), lambda j: (0, j)),
        out_shape=jax.ShapeDtypeStruct((1, B), jnp.float32),
    )(w1t, embt, b1, w2t, b2)


def kernel(states, emb_table, W1, b1, W2, b2):
    idx = states.reshape(NW, NCH, CHUNK)
    emb_t = _sc_gather_t(idx, emb_table)
    values_t = _tc_mlp_t(
        W1.T, emb_t, b1.reshape(HID, 1), W2.T, b2.reshape(1, 1)
    )
    return emb_t.T, values_t.reshape(B, 1)


# speedup vs baseline: 1.0178x; 1.0178x over previous
import functools

import jax
import jax.numpy as jnp
from jax import lax
from jax.experimental import pallas as pl
from jax.experimental.pallas import tpu as pltpu
from jax.experimental.pallas import tpu_sc as plsc

B = 16384
D = 64
HID = 16

_info = plsc.get_sparse_core_info()
NC, NS = _info.num_cores, _info.num_subcores
NW = NC * NS
B_PER_W = B // NW
CHUNK = 128
NCH = B_PER_W // CHUNK

_mesh = plsc.VectorSubcoreMesh(core_axis_name="c", subcore_axis_name="s")


@functools.partial(
    pl.kernel,
    mesh=_mesh,
    out_type=jax.ShapeDtypeStruct((B, D), jnp.float32),
    scratch_types=[
        pltpu.VMEM((NCH, CHUNK), jnp.int32),
        pltpu.VMEM((B_PER_W, D), jnp.float32),
        pltpu.SemaphoreType.DMA,
    ],
    compiler_params=pltpu.CompilerParams(use_tc_tiling_on_sc=False),
)
def _sc_gather(idx_hbm, table_hbm, rows_hbm, idx_v, rows_v, sem):
    wid = lax.axis_index("s") * NC + lax.axis_index("c")
    pltpu.sync_copy(idx_hbm.at[wid], idx_v)
    handles = []
    for k in range(NCH):
        handles.append(
            pltpu.async_copy(
                table_hbm.at[idx_v.at[k]],
                rows_v.at[pl.ds(k * CHUNK, CHUNK)],
                sem,
            )
        )
    for h in handles:
        h.wait()
    pltpu.sync_copy(rows_v, rows_hbm.at[pl.ds(wid * B_PER_W, B_PER_W)])


def _mlp_body(rows_ref, w1_ref, b1_ref, w2_ref, b2_ref, val_ref, embt_ref):
    rows = rows_ref[...]
    embt_ref[...] = rows.T
    h = jnp.dot(rows, w1_ref[...], preferred_element_type=jnp.float32)
    h = jnp.maximum(h + b1_ref[...], 0.0)
    val_ref[...] = (
        jnp.dot(h, w2_ref[...], preferred_element_type=jnp.float32) + b2_ref[...]
    )


_BBLK = 2048


def _tc_mlp(rows, w1, b1, w2, b2):
    grid = (B // _BBLK,)
    return pl.pallas_call(
        _mlp_body,
        grid=grid,
        in_specs=[
            pl.BlockSpec((_BBLK, D), lambda i: (i, 0)),
            pl.BlockSpec((D, HID), lambda i: (0, 0)),
            pl.BlockSpec((1, HID), lambda i: (0, 0)),
            pl.BlockSpec((HID, 1), lambda i: (0, 0)),
            pl.BlockSpec((1, 1), lambda i: (0, 0)),
        ],
        out_specs=[
            pl.BlockSpec((_BBLK, 1), lambda i: (i, 0)),
            pl.BlockSpec((D, _BBLK), lambda i: (0, i)),
        ],
        out_shape=[
            jax.ShapeDtypeStruct((B, 1), jnp.float32),
            jax.ShapeDtypeStruct((D, B), jnp.float32),
        ],
    )(rows, w1, b1, w2, b2)


def kernel(states, emb_table, W1, b1, W2, b2):
    idx = states.reshape(NW, NCH, CHUNK)
    rows = _sc_gather(idx, emb_table)
    values, emb_t = _tc_mlp(
        rows, W1, b1.reshape(1, HID), W2, b2.reshape(1, 1)
    )
    return emb_t.T, values
